# Initial kernel scaffold; baseline (speedup 1.0000x reference)
import jax
import jax.numpy as jnp
from jax.experimental import pallas as pl


def kernel(x, edge_index, W1, b1, W2, b2):
    raise NotImplementedError("write your pallas kernel here")


# speedup vs baseline: 21.5724x; 21.5724x over previous
import functools

import jax
import jax.numpy as jnp
from jax import lax
from jax.experimental import pallas as pl
from jax.experimental.pallas import tpu as pltpu
from jax.experimental.pallas import tpu_sc as plsc

N = 10000
D = 128
H = 64
E = 320000
NGEN = 1024

NC = 2
NS = 16
NW = NC * NS
CHUNK = 128
K = 4
NPAD = 10112
RPT = NPAD // NS
EPAD = 327680
CPW = EPAD // (NW * CHUNK)
NG = CPW // K

_mesh = plsc.VectorSubcoreMesh(core_axis_name="c", subcore_axis_name="s")
_sc_params = pltpu.CompilerParams(use_tc_tiling_on_sc=False)


@functools.partial(
    pl.kernel,
    out_type=jax.ShapeDtypeStruct((NC * NPAD, H), jnp.float32),
    mesh=_mesh,
    compiler_params=_sc_params,
    scratch_types=[
        pltpu.VMEM_SHARED((NPAD, H), jnp.float32),
        pltpu.VMEM((CPW, CHUNK), jnp.int32),
        pltpu.VMEM((CPW, CHUNK), jnp.int32),
        pltpu.VMEM((2 * K, CHUNK, H), jnp.float32),
        pltpu.SemaphoreType.DMA,
        pltpu.SemaphoreType.DMA,
    ],
)
def _edge_pass64(table, src2, dst2, zrows, out, acc, sidx, didx, rbuf,
                 sem0, sem1):
  c = lax.axis_index("c")
  s = lax.axis_index("s")
  wid = c * NS + s
  pltpu.sync_copy(zrows, acc.at[pl.ds(s * RPT, RPT)])
  pltpu.sync_copy(src2.at[pl.ds(wid * CPW, CPW)], sidx)
  pltpu.sync_copy(dst2.at[pl.ds(wid * CPW, CPW)], didx)
  plsc.subcore_barrier()

  sems = (sem0, sem1)

  def fire(g, b):
    for i in range(K):
      pltpu.async_copy(table.at[sidx.at[g * K + i]], rbuf.at[b * K + i],
                       sems[b])

  def wait_scatter(g, b):
    for i in range(K):
      pltpu.make_async_copy(table.at[sidx.at[g * K + i]],
                            rbuf.at[b * K + i], sems[b]).wait()
      pltpu.sync_copy(rbuf.at[b * K + i], acc.at[didx.at[g * K + i]],
                      add=True)

  fire(0, 0)
  fire(1, 1)

  def body(p, carry):
    g0 = 2 * p
    wait_scatter(g0, 0)
    fire(g0 + 2, 0)
    wait_scatter(g0 + 1, 1)
    fire(g0 + 3, 1)
    return carry

  lax.fori_loop(0, NG // 2 - 1, body, 0)
  wait_scatter(NG - 2, 0)
  wait_scatter(NG - 1, 1)

  plsc.subcore_barrier()
  pltpu.sync_copy(acc.at[pl.ds(s * RPT, RPT)],
                  out.at[pl.ds(c * NPAD + s * RPT, RPT)])


def _edge_pass1d(length, rows):
  zlen = length // NS

  @functools.partial(
      pl.kernel,
      out_type=jax.ShapeDtypeStruct((NC * length,), jnp.float32),
      mesh=_mesh,
      compiler_params=_sc_params,
      scratch_types=[
          pltpu.VMEM_SHARED((length,), jnp.float32),
          pltpu.VMEM((rows, CHUNK), jnp.int32),
          pltpu.VMEM((rows, CHUNK), jnp.int32),
          pltpu.VMEM((2 * K, CHUNK), jnp.float32),
          pltpu.SemaphoreType.DMA,
          pltpu.SemaphoreType.DMA,
      ],
  )
  def kern(table, src2, dst2, zrows, out, acc, sidx, didx, rbuf, sem0, sem1):
    c = lax.axis_index("c")
    s = lax.axis_index("s")
    wid = c * NS + s
    pltpu.sync_copy(zrows, acc.at[pl.ds(s * zlen, zlen)])
    pltpu.sync_copy(src2.at[pl.ds(wid * rows, rows)], sidx)
    pltpu.sync_copy(dst2.at[pl.ds(wid * rows, rows)], didx)
    plsc.subcore_barrier()

    sems = (sem0, sem1)
    ng = rows // K

    def fire(g, b):
      for i in range(K):
        pltpu.async_copy(table.at[sidx.at[g * K + i]], rbuf.at[b * K + i],
                         sems[b])

    def wait_scatter(g, b):
      for i in range(K):
        pltpu.make_async_copy(table.at[sidx.at[g * K + i]],
                              rbuf.at[b * K + i], sems[b]).wait()
        pltpu.sync_copy(rbuf.at[b * K + i], acc.at[didx.at[g * K + i]],
                        add=True)

    fire(0, 0)
    fire(1, 1)

    def body(p, carry):
      g0 = 2 * p
      wait_scatter(g0, 0)
      fire(g0 + 2, 0)
      wait_scatter(g0 + 1, 1)
      fire(g0 + 3, 1)
      return carry

    lax.fori_loop(0, ng // 2 - 1, body, 0)
    wait_scatter(ng - 2, 0)
    wait_scatter(ng - 1, 1)

    plsc.subcore_barrier()
    pltpu.sync_copy(acc.at[pl.ds(s * zlen, zlen)],
                    out.at[pl.ds(c * length + s * zlen, zlen)])

  return kern


@functools.partial(
    pl.kernel,
    out_type=jax.ShapeDtypeStruct((NC * NPAD,), jnp.float32),
    mesh=_mesh,
    compiler_params=_sc_params,
    scratch_types=[
        pltpu.VMEM_SHARED((NPAD,), jnp.float32),
        pltpu.VMEM((CPW, CHUNK), jnp.int32),
        pltpu.VMEM((CHUNK,), jnp.float32),
    ],
)
def _hist_kernel(dst2, zrows, ones_h, out, acc, didx, onesv):
  c = lax.axis_index("c")
  s = lax.axis_index("s")
  wid = c * NS + s
  pltpu.sync_copy(zrows, acc.at[pl.ds(s * RPT, RPT)])
  pltpu.sync_copy(dst2.at[pl.ds(wid * CPW, CPW)], didx)
  pltpu.sync_copy(ones_h, onesv)
  plsc.subcore_barrier()

  def body(j, carry):
    pltpu.sync_copy(onesv, acc.at[didx.at[j]], add=True)
    return carry

  lax.fori_loop(0, CPW, body, 0)
  plsc.subcore_barrier()
  pltpu.sync_copy(acc.at[pl.ds(s * RPT, RPT)],
                  out.at[pl.ds(c * NPAD + s * RPT, RPT)])


def _dense1_body(x_ref, w1_ref, hp_ref, hs_ref, dis_ref):
  deg = hp_ref[0:NPAD] + hp_ref[NPAD:2 * NPAD] + 1.0
  dis = lax.rsqrt(deg)
  h = jnp.dot(x_ref[...], w1_ref[...], preferred_element_type=jnp.float32)
  dis_ref[...] = dis
  hs_ref[0:N] = h * dis[0:N]


def _dense2_body(accs_ref, hs_ref, dis_ref, b1_ref, w2_ref, zs_ref):
  acc = accs_ref[0:N] + accs_ref[NPAD:NPAD + N]
  dis = dis_ref[0:N]
  out1 = jnp.maximum(dis * (acc + hs_ref[0:N]) + b1_ref[...], 0.0)
  z = jnp.dot(out1, w2_ref[...], preferred_element_type=jnp.float32)
  zs_ref[0:N] = (dis * z)[:, 0:2]


def _final_body(acc2_ref, zs_ref, dis_ref, b2_ref, out_ref):
  a = acc2_ref[0:NGEN] + acc2_ref[NPAD:NPAD + NGEN]
  out_ref[...] = dis_ref[0:NGEN] * (a + zs_ref[0:NGEN]) + b2_ref[...]


def kernel(x, edge_index, W1, b1, W2, b2):
  src = edge_index[0].astype(jnp.int32)
  dst = edge_index[1].astype(jnp.int32)
  padi = jnp.full((EPAD - E,), N, jnp.int32)
  srcp = jnp.concatenate([src, padi])
  dstp = jnp.concatenate([dst, padi])
  src2 = srcp.reshape(EPAD // CHUNK, CHUNK)
  dst2 = dstp.reshape(EPAD // CHUNK, CHUNK)
  sint = (2 * srcp[:, None] + jnp.arange(2, dtype=jnp.int32)[None, :])
  dint = (2 * dstp[:, None] + jnp.arange(2, dtype=jnp.int32)[None, :])
  src2i = sint.reshape(2 * EPAD // CHUNK, CHUNK)
  dst2i = dint.reshape(2 * EPAD // CHUNK, CHUNK)

  z64 = jnp.zeros((RPT, H), jnp.float32)
  z1 = jnp.zeros((RPT,), jnp.float32)
  z2 = jnp.zeros((2 * NPAD // NS,), jnp.float32)
  ones1 = jnp.ones((CHUNK,), jnp.float32)

  hp1 = _hist_kernel(dst2, z1, ones1)
  hp = hp1.reshape(NC * NPAD, 1)

  hs, dis = pl.pallas_call(
      _dense1_body,
      out_shape=(jax.ShapeDtypeStruct((NPAD, H), jnp.float32),
                 jax.ShapeDtypeStruct((NPAD, 1), jnp.float32)),
  )(x, W1, hp)

  accs = _edge_pass64(hs, src2, dst2, z64)

  W2p = jnp.zeros((H, 8), jnp.float32).at[:, 0:2].set(W2)
  zs = pl.pallas_call(
      _dense2_body,
      out_shape=jax.ShapeDtypeStruct((NPAD, 2), jnp.float32),
  )(accs, hs, dis, b1, W2p)

  acc2f = _edge_pass1d(2 * NPAD, 2 * CPW)(
      zs.reshape(-1), src2i, dst2i, z2)
  acc2 = acc2f.reshape(NC * NPAD, 2)

  out = pl.pallas_call(
      _final_body,
      out_shape=jax.ShapeDtypeStruct((NGEN, 2), jnp.float32),
  )(acc2, zs, dis, b2)

  return out.reshape(-1)


# speedup vs baseline: 37.8567x; 1.7549x over previous
import functools

import jax
import jax.numpy as jnp
from jax import lax
from jax.experimental import pallas as pl
from jax.experimental.pallas import tpu as pltpu
from jax.experimental.pallas import tpu_sc as plsc

N = 10000
D = 128
H = 64
E = 320000
NGEN = 1024

NC = 2
NS = 16
NW = NC * NS
CHUNK = 128
K = 4
NPAD = 10112
RPT = NPAD // NS
EPAD = 327680
CPW = EPAD // (NW * CHUNK)
NG = CPW // K

_mesh = plsc.VectorSubcoreMesh(core_axis_name="c", subcore_axis_name="s")
_sc_params = pltpu.CompilerParams(use_tc_tiling_on_sc=False)


@functools.partial(
    pl.kernel,
    out_type=jax.ShapeDtypeStruct((NC * NPAD, H), jnp.float32),
    mesh=_mesh,
    compiler_params=_sc_params,
    scratch_types=[
        pltpu.VMEM_SHARED((NPAD, H), jnp.float32),
        pltpu.VMEM((CPW, CHUNK), jnp.int32),
        pltpu.VMEM((CPW, CHUNK), jnp.int32),
        pltpu.VMEM((2 * K, CHUNK, H), jnp.float32),
        pltpu.SemaphoreType.DMA,
        pltpu.SemaphoreType.DMA,
    ],
)
def _edge_pass64(table, src2, dst2, zrows, out, acc, sidx, didx, rbuf,
                 sem0, sem1):
  c = lax.axis_index("c")
  s = lax.axis_index("s")
  wid = c * NS + s
  pltpu.sync_copy(zrows, acc.at[pl.ds(s * RPT, RPT)])
  pltpu.sync_copy(src2.at[pl.ds(wid * CPW, CPW)], sidx)
  pltpu.sync_copy(dst2.at[pl.ds(wid * CPW, CPW)], didx)
  plsc.subcore_barrier()

  sems = (sem0, sem1)

  def fire(g, b):
    for i in range(K):
      pltpu.async_copy(table.at[sidx.at[g * K + i]], rbuf.at[b * K + i],
                       sems[b])

  def wait_scatter(g, b):
    for i in range(K):
      pltpu.make_async_copy(table.at[sidx.at[g * K + i]],
                            rbuf.at[b * K + i], sems[b]).wait()
      pltpu.sync_copy(rbuf.at[b * K + i], acc.at[didx.at[g * K + i]],
                      add=True)

  fire(0, 0)
  fire(1, 1)

  def body(p, carry):
    g0 = 2 * p
    wait_scatter(g0, 0)
    fire(g0 + 2, 0)
    wait_scatter(g0 + 1, 1)
    fire(g0 + 3, 1)
    return carry

  lax.fori_loop(0, NG // 2 - 1, body, 0)
  wait_scatter(NG - 2, 0)
  wait_scatter(NG - 1, 1)

  plsc.subcore_barrier()
  pltpu.sync_copy(acc.at[pl.ds(s * RPT, RPT)],
                  out.at[pl.ds(c * NPAD + s * RPT, RPT)])


def _edge_pass1d(length, rows):
  zlen = length // NS

  @functools.partial(
      pl.kernel,
      out_type=jax.ShapeDtypeStruct((NC * length,), jnp.float32),
      mesh=_mesh,
      compiler_params=_sc_params,
      scratch_types=[
          pltpu.VMEM_SHARED((length,), jnp.float32),
          pltpu.VMEM((rows, CHUNK), jnp.int32),
          pltpu.VMEM((rows, CHUNK), jnp.int32),
          pltpu.VMEM((2 * K, CHUNK), jnp.float32),
          pltpu.SemaphoreType.DMA,
          pltpu.SemaphoreType.DMA,
      ],
  )
  def kern(table, src2, dst2, zrows, out, acc, sidx, didx, rbuf, sem0, sem1):
    c = lax.axis_index("c")
    s = lax.axis_index("s")
    wid = c * NS + s
    pltpu.sync_copy(zrows, acc.at[pl.ds(s * zlen, zlen)])
    pltpu.sync_copy(src2.at[pl.ds(wid * rows, rows)], sidx)
    pltpu.sync_copy(dst2.at[pl.ds(wid * rows, rows)], didx)
    plsc.subcore_barrier()

    sems = (sem0, sem1)
    ng = rows // K

    def fire(g, b):
      for i in range(K):
        pltpu.async_copy(table.at[sidx.at[g * K + i]], rbuf.at[b * K + i],
                         sems[b])

    def wait_scatter(g, b):
      for i in range(K):
        pltpu.make_async_copy(table.at[sidx.at[g * K + i]],
                              rbuf.at[b * K + i], sems[b]).wait()
        pltpu.sync_copy(rbuf.at[b * K + i], acc.at[didx.at[g * K + i]],
                        add=True)

    fire(0, 0)
    fire(1, 1)

    def body(p, carry):
      g0 = 2 * p
      wait_scatter(g0, 0)
      fire(g0 + 2, 0)
      wait_scatter(g0 + 1, 1)
      fire(g0 + 3, 1)
      return carry

    lax.fori_loop(0, ng // 2 - 1, body, 0)
    wait_scatter(ng - 2, 0)
    wait_scatter(ng - 1, 1)

    plsc.subcore_barrier()
    pltpu.sync_copy(acc.at[pl.ds(s * zlen, zlen)],
                    out.at[pl.ds(c * length + s * zlen, zlen)])

  return kern


@functools.partial(
    pl.kernel,
    out_type=jax.ShapeDtypeStruct((NC * NPAD,), jnp.float32),
    mesh=_mesh,
    compiler_params=_sc_params,
    scratch_types=[
        pltpu.VMEM_SHARED((NPAD,), jnp.float32),
        pltpu.VMEM((CPW, CHUNK), jnp.int32),
        pltpu.VMEM((CHUNK,), jnp.float32),
    ],
)
def _hist_kernel(dst2, zrows, ones_h, out, acc, didx, onesv):
  c = lax.axis_index("c")
  s = lax.axis_index("s")
  wid = c * NS + s
  pltpu.sync_copy(zrows, acc.at[pl.ds(s * RPT, RPT)])
  pltpu.sync_copy(dst2.at[pl.ds(wid * CPW, CPW)], didx)
  pltpu.sync_copy(ones_h, onesv)
  plsc.subcore_barrier()

  def body(j, carry):
    pltpu.sync_copy(onesv, acc.at[didx.at[j]], add=True)
    return carry

  lax.fori_loop(0, CPW, body, 0)
  plsc.subcore_barrier()
  pltpu.sync_copy(acc.at[pl.ds(s * RPT, RPT)],
                  out.at[pl.ds(c * NPAD + s * RPT, RPT)])


def _dense1_body(x_ref, w1_ref, hp_ref, hs_ref, dis_ref):
  deg = hp_ref[0:NPAD] + hp_ref[NPAD:2 * NPAD] + 1.0
  dis = lax.rsqrt(deg)
  h = jnp.dot(x_ref[...], w1_ref[...], preferred_element_type=jnp.float32)
  dis_ref[...] = dis
  hs_ref[0:N] = h * dis[0:N]


def _dense2_body(accs_ref, hs_ref, dis_ref, b1_ref, w2_ref, zs_ref):
  acc = accs_ref[0:N] + accs_ref[NPAD:NPAD + N]
  dis = dis_ref[0:N]
  out1 = jnp.maximum(dis * (acc + hs_ref[0:N]) + b1_ref[...], 0.0)
  z = jnp.dot(out1, w2_ref[...], preferred_element_type=jnp.float32)
  zs_ref[0:N] = (dis * z)[:, 0:2]


def _final_body(acc2_ref, zs_ref, dis_ref, b2_ref, out_ref):
  a = acc2_ref[0:NGEN] + acc2_ref[NPAD:NPAD + NGEN]
  out_ref[...] = dis_ref[0:NGEN] * (a + zs_ref[0:NGEN]) + b2_ref[...]


def kernel(x, edge_index, W1, b1, W2, b2):
  src = edge_index[0].astype(jnp.int32)
  dst = edge_index[1].astype(jnp.int32)
  padi = N + jnp.arange(EPAD - E, dtype=jnp.int32) % (NPAD - N)
  srcp = jnp.concatenate([src, padi])
  dstp = jnp.concatenate([dst, padi])
  src2 = srcp.reshape(EPAD // CHUNK, CHUNK)
  dst2 = dstp.reshape(EPAD // CHUNK, CHUNK)
  sint = (2 * srcp[:, None] + jnp.arange(2, dtype=jnp.int32)[None, :])
  dint = (2 * dstp[:, None] + jnp.arange(2, dtype=jnp.int32)[None, :])
  src2i = sint.reshape(2 * EPAD // CHUNK, CHUNK)
  dst2i = dint.reshape(2 * EPAD // CHUNK, CHUNK)

  z64 = jnp.zeros((RPT, H), jnp.float32)
  z1 = jnp.zeros((RPT,), jnp.float32)
  z2 = jnp.zeros((2 * NPAD // NS,), jnp.float32)
  ones1 = jnp.ones((CHUNK,), jnp.float32)

  hp1 = _hist_kernel(dst2, z1, ones1)
  hp = hp1.reshape(NC * NPAD, 1)

  hs, dis = pl.pallas_call(
      _dense1_body,
      out_shape=(jax.ShapeDtypeStruct((NPAD, H), jnp.float32),
                 jax.ShapeDtypeStruct((NPAD, 1), jnp.float32)),
  )(x, W1, hp)

  accs = _edge_pass64(hs, src2, dst2, z64)

  W2p = jnp.zeros((H, 8), jnp.float32).at[:, 0:2].set(W2)
  zs = pl.pallas_call(
      _dense2_body,
      out_shape=jax.ShapeDtypeStruct((NPAD, 2), jnp.float32),
  )(accs, hs, dis, b1, W2p)

  acc2f = _edge_pass1d(2 * NPAD, 2 * CPW)(
      zs.reshape(-1), src2i, dst2i, z2)
  acc2 = acc2f.reshape(NC * NPAD, 2)

  out = pl.pallas_call(
      _final_body,
      out_shape=jax.ShapeDtypeStruct((NGEN, 2), jnp.float32),
  )(acc2, zs, dis, b2)

  return out.reshape(-1)
